# baseline (speedup 1.0000x reference)
import functools
import math

import jax
import jax.numpy as jnp
from jax.experimental import pallas as pl
from jax.experimental.pallas import tpu as pltpu

KEEP = 0.5
_I32_MIN = -(2 ** 31)


def _order_key(x):
    b = jax.lax.bitcast_convert_type(x, jnp.int32)
    return jnp.where(b >= 0, b, jnp.bitwise_xor(jnp.invert(b), jnp.int32(_I32_MIN)))


def _fused_kernel(k, nt, bt, h_hbm, w_ref, b_ref, p_ref, m_ref, s_ref,
                  hrow_ref, srow_ref, swide_ref, thr_ref, dma_sems):
    i = pl.program_id(0)
    p = pl.program_id(1)
    t = pl.program_id(2)

    def tile_copy(tile, slot):
        return pltpu.make_async_copy(
            h_hbm.at[i, pl.ds(tile * bt, bt), :],
            hrow_ref.at[pl.ds(tile * bt, bt), :],
            dma_sems.at[slot],
        )

    @pl.when(p == 0)
    def _phase0():
        @pl.when(t == 0)
        def _():
            tile_copy(0, 0).start()

        @pl.when(t + 1 < nt)
        def _():
            tile_copy(t + 1, (t + 1) % 2).start()

        tile_copy(t, t % 2).wait()

        h = hrow_ref[pl.ds(t * bt, bt), :].astype(jnp.bfloat16)
        s_mat = jax.lax.dot_general(h, w_ref[...], (((1,), (0,)), ((), ())),
                                    preferred_element_type=jnp.float32)
        s = s_mat[:, 0:1] + b_ref[0, 0]
        srow_ref[pl.ds(t * bt, bt), :] = s
        swide_ref[pl.ds(t * (bt // 128), bt // 128), :] = s.reshape(bt // 128, 128)
        s_ref[0] = s

        @pl.when(t == nt - 1)
        def _():
            keys = _order_key(swide_ref[...])

            def body(it, prefix_u):
                j = 31 - it
                cand_u = jnp.bitwise_or(prefix_u, jnp.left_shift(jnp.int32(1), j))
                cand_i = jnp.bitwise_xor(cand_u, jnp.int32(_I32_MIN))
                cnt = jnp.sum((keys >= cand_i).astype(jnp.int32))
                return jnp.where(cnt >= k, cand_u, prefix_u)

            prefix_u = jax.lax.fori_loop(0, 32, body, jnp.int32(0))
            thr_ref[0] = jnp.bitwise_xor(prefix_u, jnp.int32(_I32_MIN))

    @pl.when(p == 1)
    def _phase1():
        s_tile = srow_ref[pl.ds(t * bt, bt), :]
        keep = _order_key(s_tile) >= thr_ref[0]
        m_ref[0] = keep
        p_ref[0] = hrow_ref[pl.ds(t * bt, bt), :] * keep.astype(jnp.float32)
        s_ref[0] = s_tile


def _run(hidden_states, W, b, interpret=False):
    B, T, D = hidden_states.shape
    k = min(max(1, math.ceil(KEEP * T)), T)
    BT = 512
    nt = T // BT
    pruned, mask_col, scores_col = pl.pallas_call(
        functools.partial(_fused_kernel, k, nt, BT),
        grid=(B, 2, nt),
        in_specs=[
            pl.BlockSpec(memory_space=pl.ANY),
            pl.BlockSpec((D, 128), lambda i, p, j: (0, 0)),
            pl.BlockSpec((1, 1), lambda i, p, j: (0, 0)),
        ],
        out_specs=[
            pl.BlockSpec((1, BT, D), lambda i, p, j: (i, j * p, 0)),
            pl.BlockSpec((1, BT, 1), lambda i, p, j: (i, j * p, 0)),
            pl.BlockSpec((1, BT, 1), lambda i, p, j: (i, j, 0)),
        ],
        out_shape=[
            jax.ShapeDtypeStruct((B, T, D), jnp.float32),
            jax.ShapeDtypeStruct((B, T, 1), jnp.bool_),
            jax.ShapeDtypeStruct((B, T, 1), jnp.float32),
        ],
        scratch_shapes=[
            pltpu.VMEM((T, D), jnp.float32),
            pltpu.VMEM((T, 1), jnp.float32),
            pltpu.VMEM((T // 128, 128), jnp.float32),
            pltpu.SMEM((1,), jnp.int32),
            pltpu.SemaphoreType.DMA((2,)),
        ],
        interpret=interpret,
    )(hidden_states,
      jnp.broadcast_to(W.reshape(D, 1).astype(jnp.bfloat16), (D, 128)),
      b.reshape(1, 1))
    return (pruned, mask_col.reshape(B, T), scores_col.reshape(B, T))


def kernel(hidden_states, W, b, interpret=False):
    return _run(hidden_states, W, b, interpret)


# speedup vs baseline: 1.2289x; 1.2289x over previous
import functools
import math

import jax
import jax.numpy as jnp
from jax.experimental import pallas as pl
from jax.experimental.pallas import tpu as pltpu

KEEP = 0.5
_I32_MIN = -(2 ** 31)


def _order_key(x):
    b = jax.lax.bitcast_convert_type(x, jnp.int32)
    return jnp.where(b >= 0, b, jnp.bitwise_xor(jnp.invert(b), jnp.int32(_I32_MIN)))


def _fused_kernel(k, nt, bt, h_hbm, w_ref, b_ref, p_ref, m_ref, s_ref,
                  hrow_ref, swide_ref, thr_ref, dma_sems):
    i = pl.program_id(0)
    p = pl.program_id(1)
    t = pl.program_id(2)

    def tile_copy(tile, slot):
        return pltpu.make_async_copy(
            h_hbm.at[i, pl.ds(tile * bt, bt), :],
            hrow_ref.at[pl.ds(tile * bt, bt), :],
            dma_sems.at[slot],
        )

    @pl.when(p == 0)
    def _phase0():
        @pl.when(t == 0)
        def _():
            tile_copy(0, 0).start()

        @pl.when(t + 1 < nt)
        def _():
            tile_copy(t + 1, (t + 1) % 2).start()

        tile_copy(t, t % 2).wait()

        h = hrow_ref[pl.ds(t * bt, bt), :].astype(jnp.bfloat16)
        s_mat = jax.lax.dot_general(h, w_ref[...], (((1,), (0,)), ((), ())),
                                    preferred_element_type=jnp.float32)
        s = s_mat[:, 0:1] + b_ref[0, 0]
        s_ref[0, pl.ds(t * bt, bt), :] = s
        swide_ref[pl.ds(t * (bt // 128), bt // 128), :] = s.reshape(bt // 128, 128)

        @pl.when(t == nt - 1)
        def _():
            keys = _order_key(swide_ref[...])

            def body(it, prefix_u):
                j = 31 - it
                cand_u = jnp.bitwise_or(prefix_u, jnp.left_shift(jnp.int32(1), j))
                cand_i = jnp.bitwise_xor(cand_u, jnp.int32(_I32_MIN))
                cnt = jnp.sum((keys >= cand_i).astype(jnp.int32))
                return jnp.where(cnt >= k, cand_u, prefix_u)

            prefix_u = jax.lax.fori_loop(0, 32, body, jnp.int32(0))
            thr_ref[0] = jnp.bitwise_xor(prefix_u, jnp.int32(_I32_MIN))

    @pl.when(p == 1)
    def _phase1():
        s_tile = s_ref[0, pl.ds(t * bt, bt), :]
        keep = _order_key(s_tile) >= thr_ref[0]
        m_ref[0, pl.ds(t * bt, bt), :] = keep
        p_ref[0] = hrow_ref[pl.ds(t * bt, bt), :] * keep.astype(jnp.float32)


def _run(hidden_states, W, b, interpret=False):
    B, T, D = hidden_states.shape
    k = min(max(1, math.ceil(KEEP * T)), T)
    BT = 1024
    nt = T // BT
    pruned, mask_col, scores_col = pl.pallas_call(
        functools.partial(_fused_kernel, k, nt, BT),
        grid=(B, 2, nt),
        in_specs=[
            pl.BlockSpec(memory_space=pl.ANY),
            pl.BlockSpec((D, 128), lambda i, p, j: (0, 0)),
            pl.BlockSpec((1, 1), lambda i, p, j: (0, 0)),
        ],
        out_specs=[
            pl.BlockSpec((1, BT, D), lambda i, p, j: (i, j * p, 0)),
            pl.BlockSpec((1, T, 1), lambda i, p, j: (i, 0, 0)),
            pl.BlockSpec((1, T, 1), lambda i, p, j: (i, 0, 0)),
        ],
        out_shape=[
            jax.ShapeDtypeStruct((B, T, D), jnp.float32),
            jax.ShapeDtypeStruct((B, T, 1), jnp.bool_),
            jax.ShapeDtypeStruct((B, T, 1), jnp.float32),
        ],
        scratch_shapes=[
            pltpu.VMEM((T, D), jnp.float32),
            pltpu.VMEM((T // 128, 128), jnp.float32),
            pltpu.SMEM((1,), jnp.int32),
            pltpu.SemaphoreType.DMA((2,)),
        ],
        interpret=interpret,
    )(hidden_states,
      jnp.broadcast_to(W.reshape(D, 1).astype(jnp.bfloat16), (D, 128)),
      b.reshape(1, 1))
    return (pruned, mask_col.reshape(B, T), scores_col.reshape(B, T))


def kernel(hidden_states, W, b, interpret=False):
    return _run(hidden_states, W, b, interpret)


# speedup vs baseline: 1.4793x; 1.2038x over previous
import functools
import math

import jax
import jax.numpy as jnp
from jax.experimental import pallas as pl
from jax.experimental.pallas import tpu as pltpu

KEEP = 0.5
_I32_MIN = -(2 ** 31)


def _order_key(x):
    b = jax.lax.bitcast_convert_type(x, jnp.int32)
    return jnp.where(b >= 0, b, jnp.bitwise_xor(jnp.invert(b), jnp.int32(_I32_MIN)))


def _fused_kernel(k, nt, bt, nb, h_hbm, w_ref, b_ref, p_ref, m_ref, s_ref,
                  hrow_ref, srow_ref, thr_ref, dma_sems):
    i = pl.program_id(0)
    p = pl.program_id(1)
    t = pl.program_id(2)
    cur = jax.lax.rem(i, 2)
    bw = bt // 128

    def tile_copy(row, buf, tile):
        return pltpu.make_async_copy(
            h_hbm.at[row, pl.ds(tile * bt, bt), :],
            hrow_ref.at[buf, pl.ds(tile * bt, bt), :],
            dma_sems.at[buf, tile],
        )

    @pl.when(p == 0)
    def _phase0():
        @pl.when((t == 0) & (i == 0))
        def _():
            for tile in range(nt):
                tile_copy(0, 0, tile).start()

        @pl.when((t == 0) & (i + 1 < nb))
        def _():
            for tile in range(nt):
                tile_copy(i + 1, (i + 1) % 2, tile).start()

        tile_copy(i, cur, t).wait()

        h = hrow_ref[cur, pl.ds(t * bt, bt), :].astype(jnp.bfloat16)
        s_mat = jax.lax.dot_general(h, w_ref[...], (((1,), (0,)), ((), ())),
                                    preferred_element_type=jnp.float32)
        s = s_mat[:, 0:1] + b_ref[0, 0]
        srow_ref[pl.ds(t * bt, bt), :] = s
        s_ref[0, pl.ds(t * bw, bw), :] = s.reshape(bw, 128)

        @pl.when(t == nt - 1)
        def _():
            keys = _order_key(s_ref[0])

            def body(it, prefix_u):
                j = 31 - it
                cand_u = jnp.bitwise_or(prefix_u, jnp.left_shift(jnp.int32(1), j))
                cand_i = jnp.bitwise_xor(cand_u, jnp.int32(_I32_MIN))
                cnt = jnp.sum((keys >= cand_i).astype(jnp.int32))
                return jnp.where(cnt >= k, cand_u, prefix_u)

            prefix_u = jax.lax.fori_loop(0, 32, body, jnp.int32(0))
            thr_ref[0] = jnp.bitwise_xor(prefix_u, jnp.int32(_I32_MIN))

    @pl.when(p == 1)
    def _phase1():
        thr = thr_ref[0]
        keep_w = _order_key(s_ref[0, pl.ds(t * bw, bw), :]) >= thr
        m_ref[0, pl.ds(t * bw, bw), :] = keep_w
        keep = _order_key(srow_ref[pl.ds(t * bt, bt), :]) >= thr
        p_ref[0] = hrow_ref[cur, pl.ds(t * bt, bt), :] * keep.astype(jnp.float32)


def _run(hidden_states, W, b, interpret=False):
    B, T, D = hidden_states.shape
    k = min(max(1, math.ceil(KEEP * T)), T)
    BT = 512
    nt = T // BT
    TW = T // 128
    pruned, mask_w, scores_w = pl.pallas_call(
        functools.partial(_fused_kernel, k, nt, BT, B),
        grid=(B, 2, nt),
        in_specs=[
            pl.BlockSpec(memory_space=pl.ANY),
            pl.BlockSpec((D, 128), lambda i, p, j: (0, 0)),
            pl.BlockSpec((1, 1), lambda i, p, j: (0, 0)),
        ],
        out_specs=[
            pl.BlockSpec((1, BT, D), lambda i, p, j: (i, j * p, 0)),
            pl.BlockSpec((1, TW, 128), lambda i, p, j: (i, 0, 0)),
            pl.BlockSpec((1, TW, 128), lambda i, p, j: (i, 0, 0)),
        ],
        out_shape=[
            jax.ShapeDtypeStruct((B, T, D), jnp.float32),
            jax.ShapeDtypeStruct((B, TW, 128), jnp.bool_),
            jax.ShapeDtypeStruct((B, TW, 128), jnp.float32),
        ],
        scratch_shapes=[
            pltpu.VMEM((2, T, D), jnp.float32),
            pltpu.VMEM((T, 1), jnp.float32),
            pltpu.SMEM((1,), jnp.int32),
            pltpu.SemaphoreType.DMA((2, T // BT)),
        ],
        interpret=interpret,
    )(hidden_states,
      jnp.broadcast_to(W.reshape(D, 1).astype(jnp.bfloat16), (D, 128)),
      b.reshape(1, 1))
    return (pruned, mask_w.reshape(B, T), scores_w.reshape(B, T))


def kernel(hidden_states, W, b, interpret=False):
    return _run(hidden_states, W, b, interpret)


# speedup vs baseline: 1.8472x; 1.2487x over previous
import functools
import math

import jax
import jax.numpy as jnp
from jax.experimental import pallas as pl
from jax.experimental.pallas import tpu as pltpu

KEEP = 0.5
_I32_MIN = -(2 ** 31)


def _order_key(x):
    b = jax.lax.bitcast_convert_type(x, jnp.int32)
    return jnp.where(b >= 0, b, jnp.bitwise_xor(jnp.invert(b), jnp.int32(_I32_MIN)))


def _fused_kernel(k, nt, bt, nb, h_hbm, w_ref, b_ref, p_ref, m_ref, s_ref,
                  hrow_ref, kcol_ref, thr_ref, dma_sems):
    i = pl.program_id(0)
    p = pl.program_id(1)
    t = pl.program_id(2)
    cur = jax.lax.rem(i, 2)
    bw = bt // 128

    def tile_copy(row, buf, tile):
        return pltpu.make_async_copy(
            h_hbm.at[row, pl.ds(tile * bt, bt), :],
            hrow_ref.at[buf, pl.ds(tile * bt, bt), :],
            dma_sems.at[buf, tile],
        )

    @pl.when(p == 0)
    def _phase0():
        @pl.when((t == 0) & (i == 0))
        def _():
            for tile in range(nt):
                tile_copy(0, 0, tile).start()

        @pl.when((t == 0) & (i + 1 < nb))
        def _():
            for tile in range(nt):
                tile_copy(i + 1, (i + 1) % 2, tile).start()

        tile_copy(i, cur, t).wait()

        h = hrow_ref[cur, pl.ds(t * bt, bt), :].astype(jnp.bfloat16)
        s_mat = jax.lax.dot_general(h, w_ref[...], (((1,), (0,)), ((), ())),
                                    preferred_element_type=jnp.float32)
        s = s_mat[:, 0:1] + b_ref[0, 0]
        kcol_ref[pl.ds(t * bt, bt), :] = _order_key(s)
        s_ref[0, pl.ds(t * bw, bw), :] = s.reshape(bw, 128)

        @pl.when(t == nt - 1)
        def _():
            keys = _order_key(s_ref[0])

            def body(it, prefix_u):
                j = 31 - it
                cand_u = jnp.bitwise_or(prefix_u, jnp.left_shift(jnp.int32(1), j))
                cand_i = jnp.bitwise_xor(cand_u, jnp.int32(_I32_MIN))
                cnt = jnp.sum((keys >= cand_i).astype(jnp.int32))
                return jnp.where(cnt >= k, cand_u, prefix_u)

            prefix_u = jax.lax.fori_loop(0, 32, body, jnp.int32(0))
            thr_ref[0] = jnp.bitwise_xor(prefix_u, jnp.int32(_I32_MIN))

    @pl.when(p == 1)
    def _phase1():
        thr = thr_ref[0]
        keep_w = _order_key(s_ref[0, pl.ds(t * bw, bw), :]) >= thr
        m_ref[0, pl.ds(t * bw, bw), :] = keep_w
        keep = (kcol_ref[pl.ds(t * bt, bt), :] >= thr).astype(jnp.float32)
        p_ref[0] = hrow_ref[cur, pl.ds(t * bt, bt), :] * keep


def _run(hidden_states, W, b, interpret=False):
    B, T, D = hidden_states.shape
    k = min(max(1, math.ceil(KEEP * T)), T)
    BT = 1024
    nt = T // BT
    TW = T // 128
    pruned, mask_w, scores_w = pl.pallas_call(
        functools.partial(_fused_kernel, k, nt, BT, B),
        grid=(B, 2, nt),
        in_specs=[
            pl.BlockSpec(memory_space=pl.ANY),
            pl.BlockSpec((D, 128), lambda i, p, j: (0, 0)),
            pl.BlockSpec((1, 1), lambda i, p, j: (0, 0)),
        ],
        out_specs=[
            pl.BlockSpec((1, BT, D), lambda i, p, j: (i, j * p, 0)),
            pl.BlockSpec((1, TW, 128), lambda i, p, j: (i, 0, 0)),
            pl.BlockSpec((1, TW, 128), lambda i, p, j: (i, 0, 0)),
        ],
        out_shape=[
            jax.ShapeDtypeStruct((B, T, D), jnp.float32),
            jax.ShapeDtypeStruct((B, TW, 128), jnp.bool_),
            jax.ShapeDtypeStruct((B, TW, 128), jnp.float32),
        ],
        scratch_shapes=[
            pltpu.VMEM((2, T, D), jnp.float32),
            pltpu.VMEM((T, 1), jnp.int32),
            pltpu.SMEM((1,), jnp.int32),
            pltpu.SemaphoreType.DMA((2, T // BT)),
        ],
        compiler_params=pltpu.CompilerParams(
            vmem_limit_bytes=100 * 1024 * 1024,
        ),
        interpret=interpret,
    )(hidden_states,
      jnp.broadcast_to(W.reshape(D, 1).astype(jnp.bfloat16), (D, 128)),
      b.reshape(1, 1))
    return (pruned, mask_w.reshape(B, T), scores_w.reshape(B, T))


def kernel(hidden_states, W, b, interpret=False):
    return _run(hidden_states, W, b, interpret)
